# baseline (speedup 1.0000x reference)
import functools

import jax
import jax.numpy as jnp
from jax import lax
from jax.experimental import pallas as pl
from jax.experimental.pallas import tpu as pltpu
from jax.experimental.pallas import tpu_sc as plsc

REPAIR_P = 0.3
DEGRADE_P = 0.1
CRITICAL_P = 0.01

_R, _C = 4096, 4096
_N_SC = 0
_NW = 32
_SC_ROWS_PER_W = _N_SC // _NW
_CHUNK_ROWS = 8
_RGROUPS = _SC_ROWS_PER_W // _CHUNK_ROWS
_HALF = _C // 2
_TC_BLOCK_ROWS = 2048


def _update(eq, rnd, S):
    damaged = eq == 0
    pristine = eq == (S - 1)
    rep_val = jnp.where(rnd < REPAIR_P, jnp.int32(S - 1), jnp.int32(0))
    crit = jnp.logical_and(pristine, rnd < CRITICAL_P)
    nd_val = jnp.where(crit, jnp.int32(0), jnp.where(rnd < DEGRADE_P, eq - 1, eq))
    return jnp.where(damaged, rep_val, nd_val)


def _make_sc_call(S):
    mesh = plsc.VectorSubcoreMesh(core_axis_name="c", subcore_axis_name="s")

    @functools.partial(
        pl.kernel,
        out_type=jax.ShapeDtypeStruct((_N_SC, _C), jnp.int32),
        mesh=mesh,
        scratch_types=[
            pltpu.VMEM((2, _CHUNK_ROWS, _HALF), jnp.int32),
            pltpu.VMEM((2, _CHUNK_ROWS, _HALF), jnp.float32),
            pltpu.SemaphoreType.DMA,
            pltpu.SemaphoreType.DMA,
            pltpu.SemaphoreType.DMA,
            pltpu.SemaphoreType.DMA,
            pltpu.SemaphoreType.DMA,
            pltpu.SemaphoreType.DMA,
        ],
    )
    def run(eq_hbm, rnd_hbm, out_hbm, eqb, rndb,
            seq0, seq1, srnd0, srnd1, sout0, sout1):
        wid = lax.axis_index("s") * 2 + lax.axis_index("c")
        row0 = wid * _SC_ROWS_PER_W
        in_sems = ((seq0, srnd0), (seq1, srnd1))
        out_sems = (sout0, sout1)

        def hbm_slices(rg, s):
            r0 = row0 + rg * _CHUNK_ROWS
            c0 = s * _HALF
            sl = (pl.ds(r0, _CHUNK_ROWS), pl.ds(c0, _HALF))
            return eq_hbm.at[sl], rnd_hbm.at[sl], out_hbm.at[sl]

        def start_in(rg, s):
            eq_sl, rnd_sl, _ = hbm_slices(rg, s)
            pltpu.async_copy(eq_sl, eqb.at[s], in_sems[s][0])
            pltpu.async_copy(rnd_sl, rndb.at[s], in_sems[s][1])

        def compute(s):
            for r in range(_CHUNK_ROWS):

                @plsc.parallel_loop(0, _HALF, step=16, unroll=8)
                def _(i):
                    sl = pl.ds(i, 16)
                    eqb[s, r, sl] = _update(eqb[s, r, sl], rndb[s, r, sl], S)

        start_in(0, 0)
        start_in(0, 1)

        def body(rg, carry):
            for s in (0, 1):
                eq_sl, rnd_sl, out_sl = hbm_slices(rg, s)
                pltpu.make_async_copy(eq_sl, eqb.at[s], in_sems[s][0]).wait()
                pltpu.make_async_copy(rnd_sl, rndb.at[s], in_sems[s][1]).wait()
                compute(s)
                pltpu.async_copy(eqb.at[s], out_sl, out_sems[s])
                pltpu.make_async_copy(eqb.at[s], out_sl, out_sems[s]).wait()

                @pl.when(rg + 1 < _RGROUPS)
                def _():
                    start_in(rg + 1, s)

            return carry

        lax.fori_loop(0, _RGROUPS, body, 0)

    return run


def _tc_body(eq_ref, rnd_ref, out_ref, *, S):
    out_ref[...] = _update(eq_ref[...], rnd_ref[...], S)


_TC_BLOCK_COLS = 512


def _make_tc_call(S):
    n_tc_blocks = (_R - _N_SC) // _TC_BLOCK_ROWS
    n_col_blocks = _C // _TC_BLOCK_COLS
    off = _N_SC // _TC_BLOCK_ROWS
    spec = pl.BlockSpec((_TC_BLOCK_ROWS, _TC_BLOCK_COLS), lambda i, j: (i + off, j))
    return pl.pallas_call(
        functools.partial(_tc_body, S=S),
        grid=(n_tc_blocks, n_col_blocks),
        in_specs=[spec, spec],
        out_specs=spec,
        out_shape=jax.ShapeDtypeStruct((_R, _C), jnp.int32),
    )


def kernel(equipment, randomness_source, equipment_states):
    S = equipment_states.shape[0]
    if _N_SC > 0:
        sc_out = _make_sc_call(S)(equipment, randomness_source)
        tc_out = _make_tc_call(S)(equipment, randomness_source)
        return lax.dynamic_update_slice(tc_out, sc_out, (0, 0))
    return _make_tc_call(S)(equipment, randomness_source)


# speedup vs baseline: 1.0165x; 1.0165x over previous
import functools

import jax
import jax.numpy as jnp
from jax.experimental import pallas as pl

REPAIR_P = 0.3
DEGRADE_P = 0.1
CRITICAL_P = 0.01

_R, _C = 4096, 4096
_BLOCK_ROWS = 2048
_BLOCK_COLS = 1024


def _update(eq, rnd, S):
    damaged = eq == 0
    pristine = eq == (S - 1)
    rep_val = jnp.where(rnd < REPAIR_P, jnp.int32(S - 1), jnp.int32(0))
    crit = jnp.logical_and(pristine, rnd < CRITICAL_P)
    nd_val = jnp.where(crit, jnp.int32(0), jnp.where(rnd < DEGRADE_P, eq - 1, eq))
    return jnp.where(damaged, rep_val, nd_val)


def _body(eq_ref, rnd_ref, out_ref, *, S):
    out_ref[...] = _update(eq_ref[...], rnd_ref[...], S)


def kernel(equipment, randomness_source, equipment_states):
    S = equipment_states.shape[0]
    spec = pl.BlockSpec((_BLOCK_ROWS, _BLOCK_COLS), lambda i, j: (i, j))
    return pl.pallas_call(
        functools.partial(_body, S=S),
        grid=(_R // _BLOCK_ROWS, _C // _BLOCK_COLS),
        in_specs=[spec, spec],
        out_specs=spec,
        out_shape=jax.ShapeDtypeStruct((_R, _C), jnp.int32),
    )(equipment, randomness_source)
